# speedup vs baseline: 2.4785x; 2.4785x over previous
import functools

import jax
import jax.numpy as jnp
import numpy as np
from jax.experimental import pallas as pl
from jax.experimental.pallas import tpu as pltpu

MEM = 256
K = 4
SMK = 32
HID = 1024
INP = 1024
B = 8192
EPS = 1e-12
BLK = 512

NU = 2 * K + 4 * SMK
_A0, _B0, _U0A, _U1A, _U0B, _U1B = 0, K, 2 * K, 2 * K + SMK, 2 * K + 2 * SMK, 2 * K + 3 * SMK


def _build_consts():
    ms = np.zeros((NU, 16), np.float32)
    for i in range(SMK):
        ms[_U0A + i, i // 8] = 1.0
        ms[_U0B + i, 4 + i // 8] = 1.0
        ms[_U1A + i, 8] = 1.0
        ms[_U1B + i, 9] = 1.0
    mn0 = np.zeros((16, 8), np.float32)
    mn1 = np.zeros((16, 8), np.float32)
    for k in range(K):
        mn0[k, k] = 1.0
        mn0[4 + k, 4 + k] = 1.0
        mn1[8, k] = 1.0
        mn1[9, 4 + k] = 1.0
    mab = np.zeros((NU, 8), np.float32)
    for k in range(K):
        mab[_A0 + k, k] = 1.0
        mab[_B0 + k, 4 + k] = 1.0
    mcexp = np.zeros((8, 64), np.float32)
    for i in range(SMK):
        mcexp[i // 8, i] = 1.0
        mcexp[4 + i // 8, 32 + i] = 1.0
    mu0 = np.zeros((NU, 64), np.float32)
    for i in range(SMK):
        mu0[_U0A + i, i] = 1.0
        mu0[_U0B + i, 32 + i] = 1.0
    mfold = np.zeros((64, 16), np.float32)
    for i in range(SMK):
        mfold[i, i % 8] = 1.0
        mfold[32 + i, 8 + i % 8] = 1.0
    mgexp = np.zeros((16, 512), np.float32)
    mu1 = np.zeros((NU, 512), np.float32)
    for m in range(MEM):
        mgexp[m // 32, m] = 1.0
        mgexp[8 + m // 32, 256 + m] = 1.0
        mu1[_U1A + m % 32, m] = 1.0
        mu1[_U1B + m % 32, 256 + m] = 1.0
    return ms, mn0, mn1, mab, mcexp, mu0, mfold, mgexp, mu1


_CONSTS = _build_consts()


def _nru_kernel(x_ref, h0_ref, mem_ref, wx_ref, wh0_ref, wm_ref, bh_ref,
                wch_ref, wcm_ref, bc_ref,
                ms_ref, mn0_ref, mn1_ref, mab_ref, mcexp_ref, mu0_ref,
                mfold_ref, mgexp_ref, mu1_ref,
                memnew_ref, h_ref):
    f32 = jnp.float32
    dn = (((1,), (1,)), ((), ()))
    mem = mem_ref[...]
    acc = jax.lax.dot_general(x_ref[...], wx_ref[...], dn, preferred_element_type=f32)
    acc += jax.lax.dot_general(h0_ref[...], wh0_ref[...], dn, preferred_element_type=f32)
    acc += jax.lax.dot_general(mem, wm_ref[...], dn, preferred_element_type=f32)
    h = jnp.maximum(acc + bh_ref[...], 0.0)
    h_ref[...] = h

    u = jax.lax.dot_general(h, wch_ref[...], dn, preferred_element_type=f32)
    u += jax.lax.dot_general(mem, wcm_ref[...], dn, preferred_element_type=f32)
    u += bc_ref[...]

    u2 = u * u
    p = u2 * u2 * jnp.abs(u)
    s = jnp.dot(p, ms_ref[...], preferred_element_type=f32)
    n5 = jnp.dot(s, mn0_ref[...], preferred_element_type=f32) \
        * jnp.dot(s, mn1_ref[...], preferred_element_type=f32)
    n = jnp.exp2(0.2 * jnp.log2(n5))
    ab = jnp.dot(u, mab_ref[...], preferred_element_type=f32)
    coef = ab * (0.25 / jnp.maximum(n, EPS))
    cexp = jnp.dot(coef, mcexp_ref[...], preferred_element_type=f32)
    u0 = jnp.dot(u, mu0_ref[...], preferred_element_type=f32)
    g = cexp * u0
    gf = jnp.dot(g, mfold_ref[...], preferred_element_type=f32)
    gexp = jnp.dot(gf, mgexp_ref[...], preferred_element_type=f32)
    u1 = jnp.dot(u, mu1_ref[...], preferred_element_type=f32)
    prod = gexp * u1
    memnew_ref[...] = mem + (prod[:, :MEM] - prod[:, MEM:])


def kernel(x, h0, memory, W_h, b_h, W_a, b_a, W_b, b_b, W_va, b_va, W_vb, b_vb):
    wx = W_h[:, :INP]
    wh0 = W_h[:, INP:INP + HID]
    wm = W_h[:, INP + HID:]
    wc = jnp.concatenate([W_a, W_b, W_va, W_vb], axis=0)
    wch = wc[:, :HID]
    wcm = wc[:, HID:]
    bc = jnp.concatenate([b_a, b_b, b_va, b_vb])[None, :]
    bh = b_h[None, :]

    consts = [jnp.asarray(c) for c in _CONSTS]

    grid = (B // BLK,)
    row_spec = lambda cols: pl.BlockSpec((BLK, cols), lambda i: (i, 0))
    full = lambda a: pl.BlockSpec(a.shape, lambda i: (0,) * a.ndim)

    memnew, h = pl.pallas_call(
        _nru_kernel,
        grid=grid,
        in_specs=[row_spec(INP), row_spec(HID), row_spec(MEM),
                  full(wx), full(wh0), full(wm), full(bh),
                  full(wch), full(wcm), full(bc)] + [full(c) for c in consts],
        out_specs=[row_spec(MEM), row_spec(HID)],
        out_shape=[jax.ShapeDtypeStruct((B, MEM), jnp.float32),
                   jax.ShapeDtypeStruct((B, HID), jnp.float32)],
        compiler_params=pltpu.CompilerParams(
            dimension_semantics=("parallel",),
            vmem_limit_bytes=100 * 1024 * 1024,
        ),
    )(x, h0, memory, wx, wh0, wm, bh, wch, wcm, bc, *consts)
    return memnew, h


# speedup vs baseline: 2.5676x; 1.0359x over previous
import functools

import jax
import jax.numpy as jnp
import numpy as np
from jax.experimental import pallas as pl
from jax.experimental.pallas import tpu as pltpu

MEM = 256
K = 4
SMK = 32
HID = 1024
INP = 1024
B = 8192
EPS = 1e-12
BLK = 512

NU = 2 * K + 4 * SMK
_A0, _B0, _U0A, _U1A, _U0B, _U1B = 0, K, 2 * K, 2 * K + SMK, 2 * K + 2 * SMK, 2 * K + 3 * SMK


def _build_consts():
    ms = np.zeros((NU, 16), np.float32)
    for i in range(SMK):
        ms[_U0A + i, i // 8] = 1.0
        ms[_U0B + i, 4 + i // 8] = 1.0
        ms[_U1A + i, 8] = 1.0
        ms[_U1B + i, 9] = 1.0
    mn0 = np.zeros((16, 8), np.float32)
    mn1 = np.zeros((16, 8), np.float32)
    for k in range(K):
        mn0[k, k] = 1.0
        mn0[4 + k, 4 + k] = 1.0
        mn1[8, k] = 1.0
        mn1[9, 4 + k] = 1.0
    mab = np.zeros((NU, 8), np.float32)
    for k in range(K):
        mab[_A0 + k, k] = 1.0
        mab[_B0 + k, 4 + k] = 1.0
    mcexp = np.zeros((8, 64), np.float32)
    for i in range(SMK):
        mcexp[i // 8, i] = 1.0
        mcexp[4 + i // 8, 32 + i] = 1.0
    mu0 = np.zeros((NU, 64), np.float32)
    for i in range(SMK):
        mu0[_U0A + i, i] = 1.0
        mu0[_U0B + i, 32 + i] = 1.0
    mfold = np.zeros((64, 16), np.float32)
    for i in range(SMK):
        mfold[i, i % 8] = 1.0
        mfold[32 + i, 8 + i % 8] = 1.0
    mgexp = np.zeros((16, 512), np.float32)
    mu1 = np.zeros((NU, 512), np.float32)
    for m in range(MEM):
        mgexp[m // 32, m] = 1.0
        mgexp[8 + m // 32, 256 + m] = 1.0
        mu1[_U1A + m % 32, m] = 1.0
        mu1[_U1B + m % 32, 256 + m] = 1.0
    return ms, mn0, mn1, mab, mcexp, mu0, mfold, mgexp, mu1


_CONSTS = _build_consts()


def _nru_kernel(x_ref, h0_ref, mem_ref, wx_ref, wh0_ref, wm_ref, bh_ref,
                wch_ref, wcm_ref, bc_ref,
                ms_ref, mn0_ref, mn1_ref, mab_ref, mcexp_ref, mu0_ref,
                mfold_ref, mgexp_ref, mu1_ref,
                memnew_ref, h_ref):
    f32 = jnp.float32
    bf16 = jnp.bfloat16
    dn = (((1,), (1,)), ((), ()))
    mem = mem_ref[...]
    memb = mem.astype(bf16)
    acc = jax.lax.dot_general(x_ref[...].astype(bf16), wx_ref[...], dn,
                              preferred_element_type=f32)
    acc += jax.lax.dot_general(h0_ref[...].astype(bf16), wh0_ref[...], dn,
                               preferred_element_type=f32)
    acc += jax.lax.dot_general(memb, wm_ref[...], dn, preferred_element_type=f32)
    h = jnp.maximum(acc + bh_ref[...], 0.0)
    h_ref[...] = h

    u = jax.lax.dot_general(h.astype(bf16), wch_ref[...], dn, preferred_element_type=f32)
    u += jax.lax.dot_general(memb, wcm_ref[...], dn, preferred_element_type=f32)
    u += bc_ref[...]

    u2 = u * u
    p = u2 * u2 * jnp.abs(u)
    s = jnp.dot(p, ms_ref[...], preferred_element_type=f32)
    n5 = jnp.dot(s, mn0_ref[...], preferred_element_type=f32) \
        * jnp.dot(s, mn1_ref[...], preferred_element_type=f32)
    n = jnp.exp2(0.2 * jnp.log2(n5))
    ab = jnp.dot(u, mab_ref[...], preferred_element_type=f32)
    coef = ab * (0.25 / jnp.maximum(n, EPS))
    cexp = jnp.dot(coef, mcexp_ref[...], preferred_element_type=f32)
    u0 = jnp.dot(u, mu0_ref[...], preferred_element_type=f32)
    g = cexp * u0
    gf = jnp.dot(g, mfold_ref[...], preferred_element_type=f32)
    gexp = jnp.dot(gf, mgexp_ref[...], preferred_element_type=f32)
    u1 = jnp.dot(u, mu1_ref[...], preferred_element_type=f32)
    prod = gexp * u1
    memnew_ref[...] = mem + (prod[:, :MEM] - prod[:, MEM:])


def kernel(x, h0, memory, W_h, b_h, W_a, b_a, W_b, b_b, W_va, b_va, W_vb, b_vb):
    Wb = W_h.astype(jnp.bfloat16)
    wx = Wb[:, :INP]
    wh0 = Wb[:, INP:INP + HID]
    wm = Wb[:, INP + HID:]
    wc = jnp.concatenate([W_a, W_b, W_va, W_vb], axis=0).astype(jnp.bfloat16)
    wch = wc[:, :HID]
    wcm = wc[:, HID:]
    bc = jnp.concatenate([b_a, b_b, b_va, b_vb])[None, :]
    bh = b_h[None, :]

    consts = [jnp.asarray(c) for c in _CONSTS]

    grid = (B // BLK,)
    row_spec = lambda cols: pl.BlockSpec((BLK, cols), lambda i: (i, 0))
    full = lambda a: pl.BlockSpec(a.shape, lambda i: (0,) * a.ndim)

    memnew, h = pl.pallas_call(
        _nru_kernel,
        grid=grid,
        in_specs=[row_spec(INP), row_spec(HID), row_spec(MEM),
                  full(wx), full(wh0), full(wm), full(bh),
                  full(wch), full(wcm), full(bc)] + [full(c) for c in consts],
        out_specs=[row_spec(MEM), row_spec(HID)],
        out_shape=[jax.ShapeDtypeStruct((B, MEM), jnp.float32),
                   jax.ShapeDtypeStruct((B, HID), jnp.float32)],
        compiler_params=pltpu.CompilerParams(
            dimension_semantics=("parallel",),
            vmem_limit_bytes=100 * 1024 * 1024,
        ),
    )(x, h0, memory, wx, wh0, wm, bh, wch, wcm, bc, *consts)
    return memnew, h


# speedup vs baseline: 2.6515x; 1.0327x over previous
import functools

import jax
import jax.numpy as jnp
import numpy as np
from jax.experimental import pallas as pl
from jax.experimental.pallas import tpu as pltpu

MEM = 256
K = 4
SMK = 32
HID = 1024
INP = 1024
B = 8192
EPS = 1e-12
BLK = 1024

NU = 2 * K + 4 * SMK
_A0, _B0, _U0A, _U1A, _U0B, _U1B = 0, K, 2 * K, 2 * K + SMK, 2 * K + 2 * SMK, 2 * K + 3 * SMK


def _build_consts():
    ms = np.zeros((NU, 16), np.float32)
    for i in range(SMK):
        ms[_U0A + i, i // 8] = 1.0
        ms[_U0B + i, 4 + i // 8] = 1.0
        ms[_U1A + i, 8] = 1.0
        ms[_U1B + i, 9] = 1.0
    mn0 = np.zeros((16, 8), np.float32)
    mn1 = np.zeros((16, 8), np.float32)
    for k in range(K):
        mn0[k, k] = 1.0
        mn0[4 + k, 4 + k] = 1.0
        mn1[8, k] = 1.0
        mn1[9, 4 + k] = 1.0
    mab = np.zeros((NU, 8), np.float32)
    for k in range(K):
        mab[_A0 + k, k] = 1.0
        mab[_B0 + k, 4 + k] = 1.0
    mcexp = np.zeros((8, 64), np.float32)
    for i in range(SMK):
        mcexp[i // 8, i] = 1.0
        mcexp[4 + i // 8, 32 + i] = 1.0
    mu0 = np.zeros((NU, 64), np.float32)
    for i in range(SMK):
        mu0[_U0A + i, i] = 1.0
        mu0[_U0B + i, 32 + i] = 1.0
    mfold = np.zeros((64, 16), np.float32)
    for i in range(SMK):
        mfold[i, i % 8] = 1.0
        mfold[32 + i, 8 + i % 8] = 1.0
    mgexp = np.zeros((16, 512), np.float32)
    mu1 = np.zeros((NU, 512), np.float32)
    for m in range(MEM):
        mgexp[m // 32, m] = 1.0
        mgexp[8 + m // 32, 256 + m] = 1.0
        mu1[_U1A + m % 32, m] = 1.0
        mu1[_U1B + m % 32, 256 + m] = 1.0
    return ms, mn0, mn1, mab, mcexp, mu0, mfold, mgexp, mu1


_CONSTS = _build_consts()


def _nru_kernel(x_ref, h0_ref, mem_ref, wt_ref, bh_ref,
                wct_ref, bc_ref,
                ms_ref, mn0_ref, mn1_ref, mab_ref, mcexp_ref, mu0_ref,
                mfold_ref, mgexp_ref, mu1_ref,
                memnew_ref, h_ref):
    f32 = jnp.float32
    bf16 = jnp.bfloat16
    dnn = (((1,), (0,)), ((), ()))
    mem = mem_ref[...]
    memb = mem.astype(bf16)
    cin = jnp.concatenate(
        [x_ref[...].astype(bf16), h0_ref[...].astype(bf16), memb], axis=1)
    acc = jax.lax.dot_general(cin, wt_ref[...], dnn, preferred_element_type=f32)
    h = jnp.maximum(acc + bh_ref[...], 0.0)
    h_ref[...] = h

    hm = jnp.concatenate([h.astype(bf16), memb], axis=1)
    u = jax.lax.dot_general(hm, wct_ref[...], dnn, preferred_element_type=f32)
    u += bc_ref[...]

    u2 = u * u
    p = u2 * u2 * jnp.abs(u)
    s = jnp.dot(p, ms_ref[...], preferred_element_type=f32)
    n5 = jnp.dot(s, mn0_ref[...], preferred_element_type=f32) \
        * jnp.dot(s, mn1_ref[...], preferred_element_type=f32)
    n = jnp.exp2(0.2 * jnp.log2(n5))
    ab = jnp.dot(u, mab_ref[...], preferred_element_type=f32)
    coef = ab * (0.25 / jnp.maximum(n, EPS))
    cexp = jnp.dot(coef, mcexp_ref[...], preferred_element_type=f32)
    u0 = jnp.dot(u, mu0_ref[...], preferred_element_type=f32)
    g = cexp * u0
    gf = jnp.dot(g, mfold_ref[...], preferred_element_type=f32)
    gexp = jnp.dot(gf, mgexp_ref[...], preferred_element_type=f32)
    u1 = jnp.dot(u, mu1_ref[...], preferred_element_type=f32)
    prod = gexp * u1
    memnew_ref[...] = mem + (prod[:, :MEM] - prod[:, MEM:])


def kernel(x, h0, memory, W_h, b_h, W_a, b_a, W_b, b_b, W_va, b_va, W_vb, b_vb):
    wt = W_h.T.astype(jnp.bfloat16)
    wct = jnp.concatenate([W_a, W_b, W_va, W_vb],
                          axis=0).T.astype(jnp.bfloat16)
    bc = jnp.concatenate([b_a, b_b, b_va, b_vb])[None, :]
    bh = b_h[None, :]

    consts = [jnp.asarray(c) for c in _CONSTS]

    grid = (B // BLK,)
    row_spec = lambda cols: pl.BlockSpec((BLK, cols), lambda i: (i, 0))
    full = lambda a: pl.BlockSpec(a.shape, lambda i: (0,) * a.ndim)

    memnew, h = pl.pallas_call(
        _nru_kernel,
        grid=grid,
        in_specs=[row_spec(INP), row_spec(HID), row_spec(MEM),
                  full(wt), full(bh),
                  full(wct), full(bc)] + [full(c) for c in consts],
        out_specs=[row_spec(MEM), row_spec(HID)],
        out_shape=[jax.ShapeDtypeStruct((B, MEM), jnp.float32),
                   jax.ShapeDtypeStruct((B, HID), jnp.float32)],
        compiler_params=pltpu.CompilerParams(
            dimension_semantics=("parallel",),
            vmem_limit_bytes=100 * 1024 * 1024,
        ),
    )(x, h0, memory, wt, bh, wct, bc, *consts)
    return memnew, h


# speedup vs baseline: 2.8419x; 1.0718x over previous
import functools

import jax
import jax.numpy as jnp
import numpy as np
from jax.experimental import pallas as pl
from jax.experimental.pallas import tpu as pltpu

MEM = 256
K = 4
SMK = 32
HID = 1024
INP = 1024
B = 8192
EPS = 1e-12
BLK = 1024

NU = 2 * K + 4 * SMK
_A0, _B0, _U0A, _U1A, _U0B, _U1B = 0, K, 2 * K, 2 * K + SMK, 2 * K + 2 * SMK, 2 * K + 3 * SMK


def _build_consts():
    ms = np.zeros((NU, 16), np.float32)
    for i in range(SMK):
        ms[_U0A + i, i // 8] = 1.0
        ms[_U0B + i, 4 + i // 8] = 1.0
        ms[_U1A + i, 8] = 1.0
        ms[_U1B + i, 9] = 1.0
    mn0 = np.zeros((16, 8), np.float32)
    mn1 = np.zeros((16, 8), np.float32)
    for k in range(K):
        mn0[k, k] = 1.0
        mn0[4 + k, 4 + k] = 1.0
        mn1[8, k] = 1.0
        mn1[9, 4 + k] = 1.0
    mab = np.zeros((NU, 8), np.float32)
    for k in range(K):
        mab[_A0 + k, k] = 1.0
        mab[_B0 + k, 4 + k] = 1.0
    mcexp = np.zeros((8, 64), np.float32)
    for i in range(SMK):
        mcexp[i // 8, i] = 1.0
        mcexp[4 + i // 8, 32 + i] = 1.0
    mu0 = np.zeros((NU, 64), np.float32)
    for i in range(SMK):
        mu0[_U0A + i, i] = 1.0
        mu0[_U0B + i, 32 + i] = 1.0
    mfold = np.zeros((64, 16), np.float32)
    for i in range(SMK):
        mfold[i, i % 8] = 1.0
        mfold[32 + i, 8 + i % 8] = 1.0
    mgexp = np.zeros((16, 512), np.float32)
    mu1 = np.zeros((NU, 512), np.float32)
    for m in range(MEM):
        mgexp[m // 32, m] = 1.0
        mgexp[8 + m // 32, 256 + m] = 1.0
        mu1[_U1A + m % 32, m] = 1.0
        mu1[_U1B + m % 32, 256 + m] = 1.0
    return ms, mn0, mn1, mab, mcexp, mu0, mfold, mgexp, mu1


_CONSTS = _build_consts()


def _nru_kernel(x_ref, h0_ref, mem_ref, wh_ref, bh_ref,
                wct_ref, bc_ref,
                ms_ref, mn0_ref, mn1_ref, mab_ref, mcexp_ref, mu0_ref,
                mfold_ref, mgexp_ref, mu1_ref,
                memnew_ref, h_ref, wbf_ref):
    f32 = jnp.float32
    bf16 = jnp.bfloat16
    dnn = (((1,), (0,)), ((), ()))
    dnt = (((1,), (1,)), ((), ()))

    @pl.when(pl.program_id(0) == 0)
    def _cast_weights():
        wbf_ref[...] = wh_ref[...].astype(bf16)

    mem = mem_ref[...]
    memb = mem.astype(bf16)
    cin = jnp.concatenate(
        [x_ref[...].astype(bf16), h0_ref[...].astype(bf16), memb], axis=1)
    acc = jax.lax.dot_general(cin, wbf_ref[...], dnt, preferred_element_type=f32)
    h = jnp.maximum(acc + bh_ref[...], 0.0)
    h_ref[...] = h

    hm = jnp.concatenate([h.astype(bf16), memb], axis=1)
    u = jax.lax.dot_general(hm, wct_ref[...], dnn, preferred_element_type=f32)
    u += bc_ref[...]

    u2 = u * u
    p = u2 * u2 * jnp.abs(u)
    s = jnp.dot(p, ms_ref[...], preferred_element_type=f32)
    n5 = jnp.dot(s, mn0_ref[...], preferred_element_type=f32) \
        * jnp.dot(s, mn1_ref[...], preferred_element_type=f32)
    n = jnp.exp2(0.2 * jnp.log2(n5))
    ab = jnp.dot(u, mab_ref[...], preferred_element_type=f32)
    coef = ab * (0.25 / jnp.maximum(n, EPS))
    cexp = jnp.dot(coef, mcexp_ref[...], preferred_element_type=f32)
    u0 = jnp.dot(u, mu0_ref[...], preferred_element_type=f32)
    g = cexp * u0
    gf = jnp.dot(g, mfold_ref[...], preferred_element_type=f32)
    gexp = jnp.dot(gf, mgexp_ref[...], preferred_element_type=f32)
    u1 = jnp.dot(u, mu1_ref[...], preferred_element_type=f32)
    prod = gexp * u1
    memnew_ref[...] = mem + (prod[:, :MEM] - prod[:, MEM:])


def kernel(x, h0, memory, W_h, b_h, W_a, b_a, W_b, b_b, W_va, b_va, W_vb, b_vb):
    wct = jnp.concatenate([W_a, W_b, W_va, W_vb],
                          axis=0).T.astype(jnp.bfloat16)
    bc = jnp.concatenate([b_a, b_b, b_va, b_vb])[None, :]
    bh = b_h[None, :]

    consts = [jnp.asarray(c) for c in _CONSTS]

    grid = (B // BLK,)
    row_spec = lambda cols: pl.BlockSpec((BLK, cols), lambda i: (i, 0))
    full = lambda a: pl.BlockSpec(a.shape, lambda i: (0,) * a.ndim)

    memnew, h = pl.pallas_call(
        _nru_kernel,
        grid=grid,
        in_specs=[row_spec(INP), row_spec(HID), row_spec(MEM),
                  full(W_h), full(bh),
                  full(wct), full(bc)] + [full(c) for c in consts],
        out_specs=[row_spec(MEM), row_spec(HID)],
        out_shape=[jax.ShapeDtypeStruct((B, MEM), jnp.float32),
                   jax.ShapeDtypeStruct((B, HID), jnp.float32)],
        scratch_shapes=[pltpu.VMEM((HID, INP + HID + MEM), jnp.bfloat16)],
        compiler_params=pltpu.CompilerParams(
            dimension_semantics=("arbitrary",),
            vmem_limit_bytes=100 * 1024 * 1024,
        ),
    )(x, h0, memory, W_h, bh, wct, bc, *consts)
    return memnew, h
